# Initial kernel scaffold; baseline (speedup 1.0000x reference)
import jax
import jax.numpy as jnp
from jax.experimental import pallas as pl


def kernel(input_ids, wte):
    raise NotImplementedError("write your pallas kernel here")


# speedup vs baseline: 1.6485x; 1.6485x over previous
import functools

import jax
import jax.numpy as jnp
from jax import lax
from jax.experimental import pallas as pl
from jax.experimental.pallas import tpu as pltpu
from jax.experimental.pallas import tpu_sc as plsc

NC = 2
NS = 16
NW = NC * NS

C = 32
NBUF = 3


@functools.partial(jax.jit, static_argnums=())
def _embedding_call(wte, idx3):
    NWk, NCHUNK, Ck = idx3.shape
    V, D = wte.shape
    B_total = NWk * NCHUNK * Ck
    b_per_w = NCHUNK * Ck

    mesh = plsc.VectorSubcoreMesh(
        core_axis_name="c", subcore_axis_name="s", num_cores=NC, num_subcores=NS
    )

    @functools.partial(
        pl.kernel,
        out_type=jax.ShapeDtypeStruct((B_total, D), jnp.float32),
        mesh=mesh,
        scratch_types=[
            pltpu.VMEM((NCHUNK, Ck), jnp.int32),
            pltpu.VMEM((NBUF, Ck, D), jnp.float32),
        ]
        + [pltpu.SemaphoreType.DMA] * (2 * NBUF),
    )
    def body(wte_h, idx_h, out_h, idx_v, bufs, *sems):
        s_in = sems[:NBUF]
        s_out = sems[NBUF:]
        cid = lax.axis_index("c")
        sid = lax.axis_index("s")
        wid = sid * NC + cid
        base = wid * b_per_w

        pltpu.sync_copy(idx_h.at[wid], idx_v)

        in_h = [None] * NBUF
        out_handle = [None] * NBUF
        for b in range(min(NBUF, NCHUNK)):
            in_h[b] = pltpu.async_copy(wte_h.at[idx_v.at[b]], bufs.at[b], s_in[b])
        for g in range(NCHUNK):
            b = g % NBUF
            in_h[b].wait()
            out_handle[b] = pltpu.async_copy(
                bufs.at[b], out_h.at[pl.ds(base + g * Ck, Ck)], s_out[b]
            )
            ng = g + NBUF
            if ng < NCHUNK:
                out_handle[b].wait()
                in_h[b] = pltpu.async_copy(
                    wte_h.at[idx_v.at[ng]], bufs.at[b], s_in[b]
                )
        for g in range(max(0, NCHUNK - NBUF), NCHUNK):
            out_handle[g % NBUF].wait()

    return body(wte, idx3)


def kernel(input_ids, wte):
    in_shape = input_ids.shape
    D = wte.shape[1]
    ids = input_ids.reshape(-1).astype(jnp.int32)
    b_per_w = ids.shape[0] // NW
    idx3 = ids.reshape(NW, b_per_w // C, C)
    out = _embedding_call(wte, idx3)
    return out.reshape(in_shape[0], in_shape[-1], D)


# speedup vs baseline: 1.6556x; 1.0043x over previous
import functools

import jax
import jax.numpy as jnp
from jax import lax
from jax.experimental import pallas as pl
from jax.experimental.pallas import tpu as pltpu
from jax.experimental.pallas import tpu_sc as plsc

NC = 2
NS = 16
NW = NC * NS

C = 16
NBUF = 7


@functools.partial(jax.jit, static_argnums=())
def _embedding_call(wte, idx3):
    NWk, NCHUNK, Ck = idx3.shape
    V, D = wte.shape
    B_total = NWk * NCHUNK * Ck
    b_per_w = NCHUNK * Ck

    mesh = plsc.VectorSubcoreMesh(
        core_axis_name="c", subcore_axis_name="s", num_cores=NC, num_subcores=NS
    )

    @functools.partial(
        pl.kernel,
        out_type=jax.ShapeDtypeStruct((B_total, D), jnp.float32),
        mesh=mesh,
        scratch_types=[
            pltpu.VMEM((NCHUNK, Ck), jnp.int32),
            pltpu.VMEM((NBUF, Ck, D), jnp.float32),
        ]
        + [pltpu.SemaphoreType.DMA] * (2 * NBUF),
    )
    def body(wte_h, idx_h, out_h, idx_v, bufs, *sems):
        s_in = sems[:NBUF]
        s_out = sems[NBUF:]
        cid = lax.axis_index("c")
        sid = lax.axis_index("s")
        wid = sid * NC + cid
        base = wid * b_per_w

        pltpu.sync_copy(idx_h.at[wid], idx_v)

        in_h = [None] * NBUF
        out_handle = [None] * NBUF
        for b in range(min(NBUF, NCHUNK)):
            in_h[b] = pltpu.async_copy(wte_h.at[idx_v.at[b]], bufs.at[b], s_in[b])
        for g in range(NCHUNK):
            b = g % NBUF
            in_h[b].wait()
            out_handle[b] = pltpu.async_copy(
                bufs.at[b], out_h.at[pl.ds(base + g * Ck, Ck)], s_out[b]
            )
            ng = g + NBUF
            if ng < NCHUNK:
                out_handle[b].wait()
                in_h[b] = pltpu.async_copy(
                    wte_h.at[idx_v.at[ng]], bufs.at[b], s_in[b]
                )
        for g in range(max(0, NCHUNK - NBUF), NCHUNK):
            out_handle[g % NBUF].wait()

    return body(wte, idx3)


def kernel(input_ids, wte):
    in_shape = input_ids.shape
    D = wte.shape[1]
    ids = input_ids.reshape(-1).astype(jnp.int32)
    b_per_w = ids.shape[0] // NW
    idx3 = ids.reshape(NW, b_per_w // C, C)
    out = _embedding_call(wte, idx3)
    return out.reshape(in_shape[0], in_shape[-1], D)
